# baseline (speedup 1.0000x reference)
import functools

import jax
import jax.numpy as jnp
from jax import lax
from jax.scipy.linalg import block_diag
from jax.experimental import pallas as pl
from jax.experimental.pallas import tpu as pltpu
from jax.experimental.pallas import tpu_sc as plsc

AX = ("xx", "yy", "zz")
D = 192
F32 = jnp.float32


def _bd3(ws):
    return block_diag(*ws)


def _cat3(bs):
    return jnp.concatenate(bs, axis=-1)


def _dot(a, b):
    return jnp.dot(a, b, preferred_element_type=F32,
                   precision=lax.Precision.DEFAULT)


def _mlp_prelu_jnp(p, x):
    for i in range(2):
        x = x @ p["layers"][i]["W"] + p["layers"][i]["b"]
        a = p["prelu"][i]
        x = jnp.where(x >= 0, x, a * x)
    return x @ p["layers"][2]["W"] + p["layers"][2]["b"]


def _full_spec(shape):
    return pl.BlockSpec(shape, lambda *_: tuple(0 for _ in shape))


def _enc_mlp(x, W0, b0, a0, W1, b1, a1, W2, b2, blk, tile3):
    R, din = x.shape
    dout = W2.shape[1] * (3 if tile3 else 1)

    def body(xr, W0r, b0r, a0r, W1r, b1r, a1r, W2r, b2r, outr):
        t = _dot(xr[...], W0r[...]) + b0r[...]
        t = jnp.where(t >= 0, t, a0r[...] * t)
        t = _dot(t, W1r[...]) + b1r[...]
        t = jnp.where(t >= 0, t, a1r[...] * t)
        h = _dot(t, W2r[...]) + b2r[...]
        if tile3:
            h = jnp.concatenate([h, h, h], axis=1)
        outr[...] = h

    grid = (R // blk,)
    return pl.pallas_call(
        body,
        grid=grid,
        in_specs=[pl.BlockSpec((blk, din), lambda n: (n, 0))]
        + [_full_spec(w.shape) for w in (W0, b0, a0, W1, b1, a1, W2, b2)],
        out_specs=pl.BlockSpec((blk, dout), lambda n: (n, 0)),
        out_shape=jax.ShapeDtypeStruct((R, dout), F32),
    )(x, W0, b0, a0, W1, b1, a1, W2, b2)


def _node_pre(h, vcat, Wsnd, Wrcv, blk):
    N = h.shape[0]

    def body(hr, vr, Wsr, Wrr, hpo, p2o):
        hp = hr[...] + vr[...]
        hpo[...] = hp
        p2o[0] = _dot(hp, Wsr[...])
        p2o[1] = _dot(hp, Wrr[...])

    return pl.pallas_call(
        body,
        grid=(N // blk,),
        in_specs=[
            pl.BlockSpec((blk, D), lambda n: (n, 0)),
            _full_spec((1, D)),
            _full_spec((D, D)),
            _full_spec((D, D)),
        ],
        out_specs=[
            pl.BlockSpec((blk, D), lambda n: (n, 0)),
            pl.BlockSpec((2, blk, D), lambda n: (0, n, 0)),
        ],
        out_shape=[
            jax.ShapeDtypeStruct((N, D), F32),
            jax.ShapeDtypeStruct((2, N, D), F32),
        ],
    )(h, vcat, Wsnd, Wrcv)


def _edge_step(pre2, ea, W1e, b1, W2, b2, W3, b3, first, blk, enc=None):
    E = pre2.shape[1]
    ea_w = ea.shape[1]
    enc = enc or []

    def body(p2r, ear, W1r, b1r, W2r, b2r, W3r, b3r, *rest):
        if first:
            (EW0, eb0, al0, EW1, eb1, al1, EW2, eb2), outs = rest[:8], rest[8:]
            t = _dot(ear[...], EW0[...]) + eb0[...]
            t = jnp.where(t >= 0, t, al0[...] * t)
            t = _dot(t, EW1[...]) + eb1[...]
            t = jnp.where(t >= 0, t, al1[...] * t)
            e64 = _dot(t, EW2[...]) + eb2[...]
            eac = jnp.concatenate([e64, e64, e64], axis=1)
        else:
            outs = rest
            eac = ear[...]
        z = (_dot(eac, W1r[...]) + p2r[0].astype(F32) + p2r[1].astype(F32)
             + b1r[...])
        z = jnp.maximum(z, 0.0)
        z = jnp.maximum(_dot(z, W2r[...]) + b2r[...], 0.0)
        m = _dot(z, W3r[...]) + b3r[...]
        outs[0][...] = m
        if first:
            outs[1][...] = eac + m

    if first:
        out_specs = [pl.BlockSpec((blk, D), lambda n: (n, 0)),
                     pl.BlockSpec((blk, D), lambda n: (n, 0))]
        out_shape = [jax.ShapeDtypeStruct((E, D), F32),
                     jax.ShapeDtypeStruct((E, D), F32)]
    else:
        out_specs = pl.BlockSpec((blk, D), lambda n: (n, 0))
        out_shape = jax.ShapeDtypeStruct((E, D), F32)
    return pl.pallas_call(
        body,
        grid=(E // blk,),
        in_specs=[
            pl.BlockSpec((2, blk, D), lambda n: (0, n, 0)),
            pl.BlockSpec((blk, ea_w), lambda n: (n, 0)),
            _full_spec((D, D)),
            _full_spec((1, D)),
            _full_spec((D, D)),
            _full_spec((1, D)),
            _full_spec((D, D)),
            _full_spec((1, D)),
        ] + [_full_spec(w.shape) for w in enc],
        out_specs=out_specs,
        out_shape=out_shape,
    )(pre2, ea, W1e, b1, W2, b2, W3, b3, *enc)


def _node_update(hp, acc0, G1h, gub, G2, gb2, G3, gb3, N1h, N1r, N1u,
                 nb1, N2, nb2, N3, nb3, blk):
    N = hp.shape[0]

    def body(hpr, acc0r, G1hr, gubr, G2r, gb2r, G3r, gb3r, N1hr, N1rr,
             N1ur, nb1r, N2r, nb2r, N3r, nb3r, hno, poo):
        pid = pl.program_id(0)
        hpv = hpr[...]
        rm = (acc0r[0, 0] - acc0r[0, 1]) + (acc0r[1, 0] - acc0r[1, 1])
        zg = jnp.maximum(_dot(hpv, G1hr[...]) + gubr[...], 0.0)
        zg = jnp.maximum(_dot(zg, G2r[...]) + gb2r[...], 0.0)
        mu = _dot(zg, G3r[...]) + gb3r[...]
        x1 = _dot(hpv, N1hr[...]) + _dot(rm, N1rr[...]) + _dot(mu, N1ur[...])
        x1 = jnp.maximum(x1 + nb1r[...], 0.0)
        x2 = jnp.maximum(_dot(x1, N2r[...]) + nb2r[...], 0.0)
        hn = hpv + _dot(x2, N3r[...]) + nb3r[...]
        hno[...] = hn
        part = jnp.sum(hn, axis=0, keepdims=True)

        @pl.when(pid == 0)
        def _():
            poo[...] = part

        @pl.when(pid != 0)
        def _():
            poo[...] = poo[...] + part

    return pl.pallas_call(
        body,
        grid=(N // blk,),
        in_specs=[
            pl.BlockSpec((blk, D), lambda n: (n, 0)),
            pl.BlockSpec((2, 2, blk, D), lambda n: (0, 0, n, 0)),
            _full_spec((D, D)),
            _full_spec((1, D)),
            _full_spec((D, D)),
            _full_spec((1, D)),
            _full_spec((D, D)),
            _full_spec((1, D)),
            _full_spec((D, D)),
            _full_spec((D, D)),
            _full_spec((D, D)),
            _full_spec((1, D)),
            _full_spec((D, D)),
            _full_spec((1, D)),
            _full_spec((D, D)),
            _full_spec((1, D)),
        ],
        out_specs=[
            pl.BlockSpec((blk, D), lambda n: (n, 0)),
            pl.BlockSpec((1, D), lambda n: (0, 0)),
        ],
        out_shape=[
            jax.ShapeDtypeStruct((N, D), F32),
            jax.ShapeDtypeStruct((1, D), F32),
        ],
    )(hp, acc0, G1h, gub, G2, gb2, G3, gb3, N1h, N1r, N1u, nb1, N2,
      nb2, N3, nb3)


def _decoder(hr, L0, db0, al0, L1, db1, al1, L2, db2, blk):
    M = hr.shape[0]
    dout = L2.shape[1]

    def body(hrr, L0r, db0r, al0r, L1r, db1r, al1r, L2r, db2r, outr):
        t = _dot(hrr[...], L0r[...]) + db0r[...]
        t = jnp.where(t >= 0, t, al0r[...] * t)
        t = _dot(t, L1r[...]) + db1r[...]
        t = jnp.where(t >= 0, t, al1r[...] * t)
        outr[...] = _dot(t, L2r[...]) + db2r[...]

    return pl.pallas_call(
        body,
        grid=(M // blk,),
        in_specs=[pl.BlockSpec((blk, D), lambda n: (n, 0))]
        + [_full_spec(w.shape) for w in (L0, db0, al0, L1, db1, al1, L2, db2)],
        out_specs=pl.BlockSpec((blk, dout), lambda n: (n, 0)),
        out_shape=jax.ShapeDtypeStruct((M, dout), F32),
    )(hr, L0, db0, al0, L1, db1, al1, L2, db2)


def _sc_gather(table, idx2d, chunk):
    n_ch_tot, chunk_ = idx2d.shape
    assert chunk_ == chunk
    M = n_ch_tot * chunk
    width = table.shape[1]
    nw = 32
    per_w = M // nw
    n_chunks = per_w // chunk
    assert per_w * nw == M and n_chunks * chunk == per_w and n_chunks >= 4
    NB = 4
    nquads = (n_chunks // NB) * NB
    tail = n_chunks - nquads
    mesh = plsc.VectorSubcoreMesh(core_axis_name="c", subcore_axis_name="s")

    @functools.partial(
        pl.kernel,
        mesh=mesh,
        out_type=jax.ShapeDtypeStruct((M, width), table.dtype),
        compiler_params=pltpu.CompilerParams(use_tc_tiling_on_sc=False),
        scratch_types=[
            pltpu.VMEM((n_chunks, chunk), jnp.int32),
        ]
        + [pltpu.VMEM((chunk, width), table.dtype) for _ in range(NB)]
        + [pltpu.SemaphoreType.DMA for _ in range(2 * NB)],
    )
    def k(t_hbm, i_hbm, o_hbm, idx_v, *bufs_sems):
        rbuf = bufs_sems[:NB]
        gs = bufs_sems[NB:2 * NB]
        ws = bufs_sems[2 * NB:]
        wid = lax.axis_index("s") * 2 + lax.axis_index("c")
        base = wid * per_w
        c0 = wid * n_chunks
        pltpu.sync_copy(i_hbm.at[pl.ds(c0, n_chunks)], idx_v)
        for j in range(NB):
            pltpu.async_copy(t_hbm.at[idx_v.at[j]], rbuf[j], gs[j])

        @pl.loop(0, nquads, step=NB)
        def _(ci):
            for j in range(NB):
                off = base + (ci + j) * chunk
                pltpu.make_async_copy(
                    t_hbm.at[idx_v.at[ci + j]], rbuf[j], gs[j]).wait()
                pltpu.async_copy(rbuf[j], o_hbm.at[pl.ds(off, chunk)], ws[j])
            for j in range(NB):
                off = base + (ci + j) * chunk
                pltpu.make_async_copy(
                    rbuf[j], o_hbm.at[pl.ds(off, chunk)], ws[j]).wait()

                @pl.when(ci + NB + j < n_chunks)
                def _():
                    pltpu.async_copy(
                        t_hbm.at[idx_v.at[ci + NB + j]], rbuf[j], gs[j])

        for j in range(tail):
            cj = nquads + j
            off = base + cj * chunk
            pltpu.make_async_copy(
                t_hbm.at[idx_v.at[cj]], rbuf[j], gs[j]).wait()
            pltpu.sync_copy(rbuf[j], o_hbm.at[pl.ds(off, chunk)])

    return k(table, idx2d)


def _sc_scatter(m0, m1, idx3, zeros, chunk):
    E = 2 * m0.shape[0]
    N = zeros.shape[0]
    DH = D // 4
    n_sub = 16
    per_t = E // (2 * n_sub)
    n_chunks = per_t // chunk
    rpt = N // n_sub
    assert per_t * 2 * n_sub == E and n_chunks * chunk == per_t
    assert rpt * n_sub == N and zeros.shape[1] == DH
    NB = 4
    nquads = (n_chunks // NB) * NB
    tail = n_chunks - nquads
    mesh = plsc.VectorSubcoreMesh(core_axis_name="c", subcore_axis_name="s")

    @functools.partial(
        pl.kernel,
        mesh=mesh,
        out_type=jax.ShapeDtypeStruct((2, 2, N, D), F32),
        compiler_params=pltpu.CompilerParams(use_tc_tiling_on_sc=False),
        scratch_types=[
            pltpu.VMEM((n_chunks, chunk), jnp.int32),
            pltpu.VMEM((n_chunks, chunk), jnp.int32),
            pltpu.VMEM_SHARED((N, DH), F32),
            pltpu.VMEM_SHARED((N, DH), F32),
        ]
        + [pltpu.VMEM((chunk, DH), F32) for _ in range(NB)]
        + [pltpu.SemaphoreType.DMA for _ in range(NB)],
    )
    def k(m0_hbm, m1_hbm, i_hbm, z_hbm, o_hbm, idxR, idxS, accP, accM,
          *bufs_sems):
        mb = bufs_sems[:NB]
        ls = bufs_sems[NB:]
        c = lax.axis_index("c")
        s = lax.axis_index("s")
        tile_e0 = s * per_t
        tile_c0 = c * (E // (2 * chunk)) + s * n_chunks
        pltpu.sync_copy(i_hbm.at[0, pl.ds(tile_c0, n_chunks)], idxR)
        pltpu.sync_copy(i_hbm.at[1, pl.ds(tile_c0, n_chunks)], idxS)

        def pipeline(m_hbm):
            def load(ci, buf, sem):
                pltpu.async_copy(
                    m_hbm.at[pl.ds(tile_e0 + ci * chunk, chunk),
                             pl.ds(col, DH)], buf, sem)

            for cp in range(4):
                col = cp * DH
                pltpu.sync_copy(z_hbm.at[pl.ds(s * rpt, rpt)],
                                accP.at[pl.ds(s * rpt, rpt)])
                pltpu.sync_copy(z_hbm.at[pl.ds(s * rpt, rpt)],
                                accM.at[pl.ds(s * rpt, rpt)])
                plsc.subcore_barrier()
                for j in range(NB):
                    load(j, mb[j], ls[j])

                @pl.loop(0, nquads, step=NB)
                def _(ci):
                    for j in range(NB):
                        pltpu.make_async_copy(
                            m_hbm.at[pl.ds(tile_e0 + (ci + j) * chunk, chunk),
                                     pl.ds(col, DH)], mb[j], ls[j]).wait()
                        pltpu.sync_copy(mb[j], accP.at[idxR.at[ci + j]],
                                        add=True)
                        pltpu.sync_copy(mb[j], accM.at[idxS.at[ci + j]],
                                        add=True)

                        @pl.when(ci + NB + j < n_chunks)
                        def _():
                            load(ci + NB + j, mb[j], ls[j])

                for j in range(tail):
                    cj = nquads + j
                    pltpu.make_async_copy(
                        m_hbm.at[pl.ds(tile_e0 + cj * chunk, chunk),
                                 pl.ds(col, DH)], mb[j], ls[j]).wait()
                    pltpu.sync_copy(mb[j], accP.at[idxR.at[cj]], add=True)
                    pltpu.sync_copy(mb[j], accM.at[idxS.at[cj]], add=True)

                plsc.subcore_barrier()
                pltpu.sync_copy(accP.at[pl.ds(s * rpt, rpt)],
                                o_hbm.at[c, 0, pl.ds(s * rpt, rpt),
                                         pl.ds(col, DH)])
                pltpu.sync_copy(accM.at[pl.ds(s * rpt, rpt)],
                                o_hbm.at[c, 1, pl.ds(s * rpt, rpt),
                                         pl.ds(col, DH)])

        @pl.when(c == 0)
        def _():
            pipeline(m0_hbm)

        @pl.when(c == 1)
        def _():
            pipeline(m1_hbm)

    return k(m0, m1, idx3, zeros)


def _step_weights(params, uenc, i):
    W1 = [params[a]["mp"][i]["edge"][0]["W"] for a in AX]
    G1 = [params[a]["mp"][i]["glob"][0]["W"] for a in AX]
    Nw1 = [params[a]["mp"][i]["node"][0]["W"] for a in AX]
    gub = _cat3([
        uenc @ G1[k][64:128] + params[AX[k]]["mp"][i]["glob"][0]["b"][None]
        for k in range(3)
    ])
    return dict(
        W1e=_bd3([w[0:64] for w in W1]),
        Wsnd=_bd3([w[64:128] for w in W1]),
        Wrcv=_bd3([w[128:192] for w in W1]),
        b1=_cat3([params[a]["mp"][i]["edge"][0]["b"][None] for a in AX]),
        W2=_bd3([params[a]["mp"][i]["edge"][1]["W"] for a in AX]),
        b2=_cat3([params[a]["mp"][i]["edge"][1]["b"][None] for a in AX]),
        W3=_bd3([params[a]["mp"][i]["edge"][2]["W"] for a in AX]),
        b3=_cat3([params[a]["mp"][i]["edge"][2]["b"][None] for a in AX]),
        G1h=_bd3([w[0:64] for w in G1]),
        gub=gub,
        G2=_bd3([params[a]["mp"][i]["glob"][1]["W"] for a in AX]),
        gb2=_cat3([params[a]["mp"][i]["glob"][1]["b"][None] for a in AX]),
        G3=_bd3([params[a]["mp"][i]["glob"][2]["W"] for a in AX]),
        gb3=_cat3([params[a]["mp"][i]["glob"][2]["b"][None] for a in AX]),
        N1h=_bd3([w[0:64] for w in Nw1]),
        N1r=_bd3([w[64:128] for w in Nw1]),
        N1u=_bd3([w[128:192] for w in Nw1]),
        nb1=_cat3([params[a]["mp"][i]["node"][0]["b"][None] for a in AX]),
        N2=_bd3([params[a]["mp"][i]["node"][1]["W"] for a in AX]),
        nb2=_cat3([params[a]["mp"][i]["node"][1]["b"][None] for a in AX]),
        N3=_bd3([params[a]["mp"][i]["node"][2]["W"] for a in AX]),
        nb3=_cat3([params[a]["mp"][i]["node"][2]["b"][None] for a in AX]),
    )


def kernel(x, edge_attr, u, params, edge_index, batch, real_nodes):
    N = x.shape[0]
    E = edge_index.shape[1]
    recv = edge_index[0]
    send = edge_index[1]

    pe = params["node_enc"]
    h = _enc_mlp(
        x,
        pe["layers"][0]["W"], pe["layers"][0]["b"][None],
        jnp.broadcast_to(pe["prelu"][0], (1, 64)),
        pe["layers"][1]["W"], pe["layers"][1]["b"][None],
        jnp.broadcast_to(pe["prelu"][1], (1, 64)),
        pe["layers"][2]["W"], pe["layers"][2]["b"][None],
        blk=2000, tile3=True)
    pe = params["edge_enc"]
    enc_w = [
        pe["layers"][0]["W"], pe["layers"][0]["b"][None],
        jnp.broadcast_to(pe["prelu"][0], (1, 64)),
        pe["layers"][1]["W"], pe["layers"][1]["b"][None],
        jnp.broadcast_to(pe["prelu"][1], (1, 64)),
        pe["layers"][2]["W"], pe["layers"][2]["b"][None],
    ]
    uenc = _mlp_prelu_jnp(params["u_enc"], u)

    v = _cat3([params[a]["vn"][0]["emb"][None, :] for a in AX])
    EH = E // 2
    idx_g = [
        jnp.concatenate([send[h * EH:(h + 1) * EH],
                         recv[h * EH:(h + 1) * EH] + N]).reshape(-1, 80)
        for h in range(2)
    ]
    idx_sc = jnp.stack([recv, send]).reshape(2, -1, 80)
    zeros = jnp.zeros((N, D // 4), F32)

    ea = [None, None]
    for i in range(2):
        w = _step_weights(params, uenc, i)
        hp, P2 = _node_pre(h, v, w["Wsnd"], w["Wrcv"], blk=2000)
        table = P2.reshape(2 * N, D)
        msgs2 = []
        for hh in range(2):
            pre2 = _sc_gather(table, idx_g[hh], chunk=80).reshape(2, EH, D)
            if i == 0:
                eah = lax.dynamic_slice_in_dim(edge_attr, hh * EH, EH, axis=0)
                msgs, ea[hh] = _edge_step(pre2, eah, w["W1e"], w["b1"],
                                          w["W2"], w["b2"], w["W3"], w["b3"],
                                          first=True, blk=2000, enc=enc_w)
            else:
                msgs = _edge_step(pre2, ea[hh], w["W1e"], w["b1"], w["W2"],
                                  w["b2"], w["W3"], w["b3"], first=False,
                                  blk=2000)
            msgs2.append(msgs)
        acc = _sc_scatter(msgs2[0], msgs2[1], idx_sc, zeros, chunk=80)
        h, pooled = _node_update(hp, acc, w["G1h"], w["gub"],
                                 w["G2"], w["gb2"], w["G3"], w["gb3"],
                                 w["N1h"], w["N1r"], w["N1u"], w["nb1"],
                                 w["N2"], w["nb2"], w["N3"], w["nb3"],
                                 blk=2000)
        vnext = []
        for k, a in enumerate(AX):
            vp = params[a]["vn"][i]
            t = pooled[:, 64 * k:64 * (k + 1)] + v[:, 64 * k:64 * (k + 1)]
            t = jnp.maximum(t @ vp["mlp"][0]["W"] + vp["mlp"][0]["b"], 0.0)
            t = jnp.maximum(t @ vp["mlp"][1]["W"] + vp["mlp"][1]["b"], 0.0)
            vnext.append(t)
        v = _cat3(vnext)

    hr = _sc_gather(h, real_nodes.reshape(-1, 64), chunk=64)
    dps = [params[n] for n in ("dx", "dy", "dz")]
    L2 = jnp.zeros((D, 128), F32)
    db2 = jnp.zeros((1, 128), F32)
    for k, p in enumerate(dps):
        L2 = L2.at[64 * k:64 * (k + 1), k].set(p["layers"][2]["W"][:, 0])
        db2 = db2.at[0, k].set(p["layers"][2]["b"][0])
    out = _decoder(
        hr,
        _bd3([p["layers"][0]["W"] for p in dps]),
        _cat3([p["layers"][0]["b"][None] for p in dps]),
        _cat3([jnp.broadcast_to(p["prelu"][0], (1, 64)) for p in dps]),
        _bd3([p["layers"][1]["W"] for p in dps]),
        _cat3([p["layers"][1]["b"][None] for p in dps]),
        _cat3([jnp.broadcast_to(p["prelu"][1], (1, 64)) for p in dps]),
        L2, db2, blk=1024)
    return out[:, :3]


# speedup vs baseline: 1.0106x; 1.0106x over previous
import functools

import jax
import jax.numpy as jnp
from jax import lax
from jax.scipy.linalg import block_diag
from jax.experimental import pallas as pl
from jax.experimental.pallas import tpu as pltpu
from jax.experimental.pallas import tpu_sc as plsc

AX = ("xx", "yy", "zz")
D = 192
F32 = jnp.float32


def _bd3(ws):
    return block_diag(*ws)


def _cat3(bs):
    return jnp.concatenate(bs, axis=-1)


def _dot(a, b):
    return jnp.dot(a, b, preferred_element_type=F32,
                   precision=lax.Precision.DEFAULT)


def _mlp_prelu_jnp(p, x):
    for i in range(2):
        x = x @ p["layers"][i]["W"] + p["layers"][i]["b"]
        a = p["prelu"][i]
        x = jnp.where(x >= 0, x, a * x)
    return x @ p["layers"][2]["W"] + p["layers"][2]["b"]


def _full_spec(shape):
    return pl.BlockSpec(shape, lambda *_: tuple(0 for _ in shape))


def _enc_mlp(x, W0, b0, a0, W1, b1, a1, W2, b2, blk, tile3):
    R, din = x.shape
    dout = W2.shape[1] * (3 if tile3 else 1)

    def body(xr, W0r, b0r, a0r, W1r, b1r, a1r, W2r, b2r, outr):
        t = _dot(xr[...], W0r[...]) + b0r[...]
        t = jnp.where(t >= 0, t, a0r[...] * t)
        t = _dot(t, W1r[...]) + b1r[...]
        t = jnp.where(t >= 0, t, a1r[...] * t)
        h = _dot(t, W2r[...]) + b2r[...]
        if tile3:
            h = jnp.concatenate([h, h, h], axis=1)
        outr[...] = h

    grid = (R // blk,)
    return pl.pallas_call(
        body,
        grid=grid,
        in_specs=[pl.BlockSpec((blk, din), lambda n: (n, 0))]
        + [_full_spec(w.shape) for w in (W0, b0, a0, W1, b1, a1, W2, b2)],
        out_specs=pl.BlockSpec((blk, dout), lambda n: (n, 0)),
        out_shape=jax.ShapeDtypeStruct((R, dout), F32),
    )(x, W0, b0, a0, W1, b1, a1, W2, b2)


def _node_pre(h, vcat, Wsnd, Wrcv, blk):
    N = h.shape[0]

    def body(hr, vr, Wsr, Wrr, hpo, p2o):
        hp = hr[...] + vr[...]
        hpo[...] = hp
        p2o[0] = _dot(hp, Wsr[...])
        p2o[1] = _dot(hp, Wrr[...])

    return pl.pallas_call(
        body,
        grid=(N // blk,),
        in_specs=[
            pl.BlockSpec((blk, D), lambda n: (n, 0)),
            _full_spec((1, D)),
            _full_spec((D, D)),
            _full_spec((D, D)),
        ],
        out_specs=[
            pl.BlockSpec((blk, D), lambda n: (n, 0)),
            pl.BlockSpec((2, blk, D), lambda n: (0, n, 0)),
        ],
        out_shape=[
            jax.ShapeDtypeStruct((N, D), F32),
            jax.ShapeDtypeStruct((2, N, D), F32),
        ],
    )(h, vcat, Wsnd, Wrcv)


def _edge_step(pre2, ea, W1e, b1, W2, b2, W3, b3, first, blk, enc=None):
    E = pre2.shape[1]
    ea_w = ea.shape[1]
    enc = enc or []

    def body(p2r, ear, W1r, b1r, W2r, b2r, W3r, b3r, *rest):
        if first:
            (EW0, eb0, al0, EW1, eb1, al1, EW2, eb2), outs = rest[:8], rest[8:]
            t = _dot(ear[...], EW0[...]) + eb0[...]
            t = jnp.where(t >= 0, t, al0[...] * t)
            t = _dot(t, EW1[...]) + eb1[...]
            t = jnp.where(t >= 0, t, al1[...] * t)
            e64 = _dot(t, EW2[...]) + eb2[...]
            eac = jnp.concatenate([e64, e64, e64], axis=1)
        else:
            outs = rest
            eac = ear[...]
        z = (_dot(eac, W1r[...]) + p2r[0].astype(F32) + p2r[1].astype(F32)
             + b1r[...])
        z = jnp.maximum(z, 0.0)
        z = jnp.maximum(_dot(z, W2r[...]) + b2r[...], 0.0)
        m = _dot(z, W3r[...]) + b3r[...]
        outs[0][...] = m
        if first:
            outs[1][...] = eac + m

    if first:
        out_specs = [pl.BlockSpec((blk, D), lambda n: (n, 0)),
                     pl.BlockSpec((blk, D), lambda n: (n, 0))]
        out_shape = [jax.ShapeDtypeStruct((E, D), F32),
                     jax.ShapeDtypeStruct((E, D), F32)]
    else:
        out_specs = pl.BlockSpec((blk, D), lambda n: (n, 0))
        out_shape = jax.ShapeDtypeStruct((E, D), F32)
    return pl.pallas_call(
        body,
        grid=(E // blk,),
        in_specs=[
            pl.BlockSpec((2, blk, D), lambda n: (0, n, 0)),
            pl.BlockSpec((blk, ea_w), lambda n: (n, 0)),
            _full_spec((D, D)),
            _full_spec((1, D)),
            _full_spec((D, D)),
            _full_spec((1, D)),
            _full_spec((D, D)),
            _full_spec((1, D)),
        ] + [_full_spec(w.shape) for w in enc],
        out_specs=out_specs,
        out_shape=out_shape,
    )(pre2, ea, W1e, b1, W2, b2, W3, b3, *enc)


def _node_update(hp, acc0, G1h, gub, G2, gb2, G3, gb3, N1h, N1r, N1u,
                 nb1, N2, nb2, N3, nb3, blk):
    N = hp.shape[0]

    def body(hpr, acc0r, G1hr, gubr, G2r, gb2r, G3r, gb3r, N1hr, N1rr,
             N1ur, nb1r, N2r, nb2r, N3r, nb3r, hno, poo):
        pid = pl.program_id(0)
        hpv = hpr[...]
        rm = (acc0r[0, 0] - acc0r[0, 1]) + (acc0r[1, 0] - acc0r[1, 1])
        zg = jnp.maximum(_dot(hpv, G1hr[...]) + gubr[...], 0.0)
        zg = jnp.maximum(_dot(zg, G2r[...]) + gb2r[...], 0.0)
        mu = _dot(zg, G3r[...]) + gb3r[...]
        x1 = _dot(hpv, N1hr[...]) + _dot(rm, N1rr[...]) + _dot(mu, N1ur[...])
        x1 = jnp.maximum(x1 + nb1r[...], 0.0)
        x2 = jnp.maximum(_dot(x1, N2r[...]) + nb2r[...], 0.0)
        hn = hpv + _dot(x2, N3r[...]) + nb3r[...]
        hno[...] = hn
        part = jnp.sum(hn, axis=0, keepdims=True)

        @pl.when(pid == 0)
        def _():
            poo[...] = part

        @pl.when(pid != 0)
        def _():
            poo[...] = poo[...] + part

    return pl.pallas_call(
        body,
        grid=(N // blk,),
        in_specs=[
            pl.BlockSpec((blk, D), lambda n: (n, 0)),
            pl.BlockSpec((2, 2, blk, D), lambda n: (0, 0, n, 0)),
            _full_spec((D, D)),
            _full_spec((1, D)),
            _full_spec((D, D)),
            _full_spec((1, D)),
            _full_spec((D, D)),
            _full_spec((1, D)),
            _full_spec((D, D)),
            _full_spec((D, D)),
            _full_spec((D, D)),
            _full_spec((1, D)),
            _full_spec((D, D)),
            _full_spec((1, D)),
            _full_spec((D, D)),
            _full_spec((1, D)),
        ],
        out_specs=[
            pl.BlockSpec((blk, D), lambda n: (n, 0)),
            pl.BlockSpec((1, D), lambda n: (0, 0)),
        ],
        out_shape=[
            jax.ShapeDtypeStruct((N, D), F32),
            jax.ShapeDtypeStruct((1, D), F32),
        ],
    )(hp, acc0, G1h, gub, G2, gb2, G3, gb3, N1h, N1r, N1u, nb1, N2,
      nb2, N3, nb3)


def _decoder(hr, L0, db0, al0, L1, db1, al1, L2, db2, blk):
    M = hr.shape[0]
    dout = L2.shape[1]

    def body(hrr, L0r, db0r, al0r, L1r, db1r, al1r, L2r, db2r, outr):
        t = _dot(hrr[...], L0r[...]) + db0r[...]
        t = jnp.where(t >= 0, t, al0r[...] * t)
        t = _dot(t, L1r[...]) + db1r[...]
        t = jnp.where(t >= 0, t, al1r[...] * t)
        outr[...] = _dot(t, L2r[...]) + db2r[...]

    return pl.pallas_call(
        body,
        grid=(M // blk,),
        in_specs=[pl.BlockSpec((blk, D), lambda n: (n, 0))]
        + [_full_spec(w.shape) for w in (L0, db0, al0, L1, db1, al1, L2, db2)],
        out_specs=pl.BlockSpec((blk, dout), lambda n: (n, 0)),
        out_shape=jax.ShapeDtypeStruct((M, dout), F32),
    )(hr, L0, db0, al0, L1, db1, al1, L2, db2)


def _sc_gather(table, idx2d, chunk):
    n_ch_tot, chunk_ = idx2d.shape
    assert chunk_ == chunk
    M = n_ch_tot * chunk
    width = table.shape[1]
    nw = 32
    per_w = M // nw
    n_chunks = per_w // chunk
    assert per_w * nw == M and n_chunks * chunk == per_w and n_chunks >= 4
    NB = 4
    nquads = (n_chunks // NB) * NB
    tail = n_chunks - nquads
    mesh = plsc.VectorSubcoreMesh(core_axis_name="c", subcore_axis_name="s")

    @functools.partial(
        pl.kernel,
        mesh=mesh,
        out_type=jax.ShapeDtypeStruct((M, width), table.dtype),
        compiler_params=pltpu.CompilerParams(use_tc_tiling_on_sc=False),
        scratch_types=[
            pltpu.VMEM((n_chunks, chunk), jnp.int32),
        ]
        + [pltpu.VMEM((chunk, width), table.dtype) for _ in range(NB)]
        + [pltpu.SemaphoreType.DMA for _ in range(2 * NB)],
    )
    def k(t_hbm, i_hbm, o_hbm, idx_v, *bufs_sems):
        rbuf = bufs_sems[:NB]
        gs = bufs_sems[NB:2 * NB]
        ws = bufs_sems[2 * NB:]
        wid = lax.axis_index("s") * 2 + lax.axis_index("c")
        base = wid * per_w
        c0 = wid * n_chunks
        pltpu.sync_copy(i_hbm.at[pl.ds(c0, n_chunks)], idx_v)
        for j in range(NB):
            pltpu.async_copy(t_hbm.at[idx_v.at[j]], rbuf[j], gs[j])

        @pl.loop(0, nquads, step=NB)
        def _(ci):
            for j in range(NB):
                off = base + (ci + j) * chunk
                pltpu.make_async_copy(
                    t_hbm.at[idx_v.at[ci + j]], rbuf[j], gs[j]).wait()
                pltpu.async_copy(rbuf[j], o_hbm.at[pl.ds(off, chunk)], ws[j])
            for j in range(NB):
                off = base + (ci + j) * chunk
                pltpu.make_async_copy(
                    rbuf[j], o_hbm.at[pl.ds(off, chunk)], ws[j]).wait()

                @pl.when(ci + NB + j < n_chunks)
                def _():
                    pltpu.async_copy(
                        t_hbm.at[idx_v.at[ci + NB + j]], rbuf[j], gs[j])

        for j in range(tail):
            cj = nquads + j
            off = base + cj * chunk
            pltpu.make_async_copy(
                t_hbm.at[idx_v.at[cj]], rbuf[j], gs[j]).wait()
            pltpu.sync_copy(rbuf[j], o_hbm.at[pl.ds(off, chunk)])

    return k(table, idx2d)


def _sc_scatter(m0, m1, idx3, zeros, chunk):
    E = 2 * m0.shape[0]
    N = zeros.shape[0]
    DH = D // 4
    n_sub = 16
    per_t = E // (2 * n_sub)
    n_chunks = per_t // chunk
    rpt = N // n_sub
    assert per_t * 2 * n_sub == E and n_chunks * chunk == per_t
    assert rpt * n_sub == N and zeros.shape[1] == DH
    NB = 4
    nquads = (n_chunks // NB) * NB
    tail = n_chunks - nquads
    mesh = plsc.VectorSubcoreMesh(core_axis_name="c", subcore_axis_name="s")

    @functools.partial(
        pl.kernel,
        mesh=mesh,
        out_type=jax.ShapeDtypeStruct((2, 2, N, D), F32),
        compiler_params=pltpu.CompilerParams(use_tc_tiling_on_sc=False),
        scratch_types=[
            pltpu.VMEM((n_chunks, chunk), jnp.int32),
            pltpu.VMEM((n_chunks, chunk), jnp.int32),
            pltpu.VMEM_SHARED((N, DH), F32),
            pltpu.VMEM_SHARED((N, DH), F32),
        ]
        + [pltpu.VMEM((chunk, DH), F32) for _ in range(NB)]
        + [pltpu.SemaphoreType.DMA for _ in range(NB)],
    )
    def k(m0_hbm, m1_hbm, i_hbm, z_hbm, o_hbm, idxR, idxS, accP, accM,
          *bufs_sems):
        mb = bufs_sems[:NB]
        ls = bufs_sems[NB:]
        c = lax.axis_index("c")
        s = lax.axis_index("s")
        tile_e0 = s * per_t
        tile_c0 = c * (E // (2 * chunk)) + s * n_chunks
        pltpu.sync_copy(i_hbm.at[0, pl.ds(tile_c0, n_chunks)], idxR)
        pltpu.sync_copy(i_hbm.at[1, pl.ds(tile_c0, n_chunks)], idxS)

        def pipeline(m_hbm):
            def load(ci, buf, sem):
                pltpu.async_copy(
                    m_hbm.at[pl.ds(tile_e0 + ci * chunk, chunk),
                             pl.ds(col, DH)], buf, sem)

            for cp in range(4):
                col = cp * DH
                pltpu.sync_copy(z_hbm.at[pl.ds(s * rpt, rpt)],
                                accP.at[pl.ds(s * rpt, rpt)])
                pltpu.sync_copy(z_hbm.at[pl.ds(s * rpt, rpt)],
                                accM.at[pl.ds(s * rpt, rpt)])
                plsc.subcore_barrier()
                for j in range(NB):
                    load(j, mb[j], ls[j])

                @pl.loop(0, nquads, step=NB)
                def _(ci):
                    for j in range(NB):
                        pltpu.make_async_copy(
                            m_hbm.at[pl.ds(tile_e0 + (ci + j) * chunk, chunk),
                                     pl.ds(col, DH)], mb[j], ls[j]).wait()
                        pltpu.sync_copy(mb[j], accP.at[idxR.at[ci + j]],
                                        add=True)
                        pltpu.sync_copy(mb[j], accM.at[idxS.at[ci + j]],
                                        add=True)

                        @pl.when(ci + NB + j < n_chunks)
                        def _():
                            load(ci + NB + j, mb[j], ls[j])

                for j in range(tail):
                    cj = nquads + j
                    pltpu.make_async_copy(
                        m_hbm.at[pl.ds(tile_e0 + cj * chunk, chunk),
                                 pl.ds(col, DH)], mb[j], ls[j]).wait()
                    pltpu.sync_copy(mb[j], accP.at[idxR.at[cj]], add=True)
                    pltpu.sync_copy(mb[j], accM.at[idxS.at[cj]], add=True)

                plsc.subcore_barrier()
                pltpu.sync_copy(accP.at[pl.ds(s * rpt, rpt)],
                                o_hbm.at[c, 0, pl.ds(s * rpt, rpt),
                                         pl.ds(col, DH)])
                pltpu.sync_copy(accM.at[pl.ds(s * rpt, rpt)],
                                o_hbm.at[c, 1, pl.ds(s * rpt, rpt),
                                         pl.ds(col, DH)])

        @pl.when(c == 0)
        def _():
            pipeline(m0_hbm)

        @pl.when(c == 1)
        def _():
            pipeline(m1_hbm)

    return k(m0, m1, idx3, zeros)


def _step_weights(params, uenc, i):
    W1 = [params[a]["mp"][i]["edge"][0]["W"] for a in AX]
    G1 = [params[a]["mp"][i]["glob"][0]["W"] for a in AX]
    Nw1 = [params[a]["mp"][i]["node"][0]["W"] for a in AX]
    gub = _cat3([
        uenc @ G1[k][64:128] + params[AX[k]]["mp"][i]["glob"][0]["b"][None]
        for k in range(3)
    ])
    return dict(
        W1e=_bd3([w[0:64] for w in W1]),
        Wsnd=_bd3([w[64:128] for w in W1]),
        Wrcv=_bd3([w[128:192] for w in W1]),
        b1=_cat3([params[a]["mp"][i]["edge"][0]["b"][None] for a in AX]),
        W2=_bd3([params[a]["mp"][i]["edge"][1]["W"] for a in AX]),
        b2=_cat3([params[a]["mp"][i]["edge"][1]["b"][None] for a in AX]),
        W3=_bd3([params[a]["mp"][i]["edge"][2]["W"] for a in AX]),
        b3=_cat3([params[a]["mp"][i]["edge"][2]["b"][None] for a in AX]),
        G1h=_bd3([w[0:64] for w in G1]),
        gub=gub,
        G2=_bd3([params[a]["mp"][i]["glob"][1]["W"] for a in AX]),
        gb2=_cat3([params[a]["mp"][i]["glob"][1]["b"][None] for a in AX]),
        G3=_bd3([params[a]["mp"][i]["glob"][2]["W"] for a in AX]),
        gb3=_cat3([params[a]["mp"][i]["glob"][2]["b"][None] for a in AX]),
        N1h=_bd3([w[0:64] for w in Nw1]),
        N1r=_bd3([w[64:128] for w in Nw1]),
        N1u=_bd3([w[128:192] for w in Nw1]),
        nb1=_cat3([params[a]["mp"][i]["node"][0]["b"][None] for a in AX]),
        N2=_bd3([params[a]["mp"][i]["node"][1]["W"] for a in AX]),
        nb2=_cat3([params[a]["mp"][i]["node"][1]["b"][None] for a in AX]),
        N3=_bd3([params[a]["mp"][i]["node"][2]["W"] for a in AX]),
        nb3=_cat3([params[a]["mp"][i]["node"][2]["b"][None] for a in AX]),
    )


def kernel(x, edge_attr, u, params, edge_index, batch, real_nodes):
    N = x.shape[0]
    E = edge_index.shape[1]
    recv = edge_index[0]
    send = edge_index[1]

    pe = params["node_enc"]
    h = _enc_mlp(
        x,
        pe["layers"][0]["W"], pe["layers"][0]["b"][None],
        jnp.broadcast_to(pe["prelu"][0], (1, 64)),
        pe["layers"][1]["W"], pe["layers"][1]["b"][None],
        jnp.broadcast_to(pe["prelu"][1], (1, 64)),
        pe["layers"][2]["W"], pe["layers"][2]["b"][None],
        blk=2000, tile3=True)
    pe = params["edge_enc"]
    enc_w = [
        pe["layers"][0]["W"], pe["layers"][0]["b"][None],
        jnp.broadcast_to(pe["prelu"][0], (1, 64)),
        pe["layers"][1]["W"], pe["layers"][1]["b"][None],
        jnp.broadcast_to(pe["prelu"][1], (1, 64)),
        pe["layers"][2]["W"], pe["layers"][2]["b"][None],
    ]
    uenc = _mlp_prelu_jnp(params["u_enc"], u)

    v = _cat3([params[a]["vn"][0]["emb"][None, :] for a in AX])
    EH = E // 2
    idx_g = [
        jnp.concatenate([send[h * EH:(h + 1) * EH],
                         recv[h * EH:(h + 1) * EH] + N]).reshape(-1, 80)
        for h in range(2)
    ]
    idx_sc = jnp.stack([recv, send]).reshape(2, -1, 80)
    zeros = jnp.zeros((N, D // 4), F32)

    ea = [None, None]
    for i in range(2):
        w = _step_weights(params, uenc, i)
        hp, P2 = _node_pre(h, v, w["Wsnd"], w["Wrcv"], blk=2000)
        table = P2.reshape(2 * N, D)
        msgs2 = []
        for hh in range(2):
            pre2 = _sc_gather(table, idx_g[hh], chunk=80).reshape(2, EH, D)
            if i == 0:
                eah = lax.dynamic_slice_in_dim(edge_attr, hh * EH, EH, axis=0)
                msgs, ea[hh] = _edge_step(pre2, eah, w["W1e"], w["b1"],
                                          w["W2"], w["b2"], w["W3"], w["b3"],
                                          first=True, blk=4000, enc=enc_w)
            else:
                msgs = _edge_step(pre2, ea[hh], w["W1e"], w["b1"], w["W2"],
                                  w["b2"], w["W3"], w["b3"], first=False,
                                  blk=4000)
            msgs2.append(msgs)
        acc = _sc_scatter(msgs2[0], msgs2[1], idx_sc, zeros, chunk=80)
        h, pooled = _node_update(hp, acc, w["G1h"], w["gub"],
                                 w["G2"], w["gb2"], w["G3"], w["gb3"],
                                 w["N1h"], w["N1r"], w["N1u"], w["nb1"],
                                 w["N2"], w["nb2"], w["N3"], w["nb3"],
                                 blk=2000)
        vnext = []
        for k, a in enumerate(AX):
            vp = params[a]["vn"][i]
            t = pooled[:, 64 * k:64 * (k + 1)] + v[:, 64 * k:64 * (k + 1)]
            t = jnp.maximum(t @ vp["mlp"][0]["W"] + vp["mlp"][0]["b"], 0.0)
            t = jnp.maximum(t @ vp["mlp"][1]["W"] + vp["mlp"][1]["b"], 0.0)
            vnext.append(t)
        v = _cat3(vnext)

    hr = _sc_gather(h, real_nodes.reshape(-1, 64), chunk=64)
    dps = [params[n] for n in ("dx", "dy", "dz")]
    L2 = jnp.zeros((D, 128), F32)
    db2 = jnp.zeros((1, 128), F32)
    for k, p in enumerate(dps):
        L2 = L2.at[64 * k:64 * (k + 1), k].set(p["layers"][2]["W"][:, 0])
        db2 = db2.at[0, k].set(p["layers"][2]["b"][0])
    out = _decoder(
        hr,
        _bd3([p["layers"][0]["W"] for p in dps]),
        _cat3([p["layers"][0]["b"][None] for p in dps]),
        _cat3([jnp.broadcast_to(p["prelu"][0], (1, 64)) for p in dps]),
        _bd3([p["layers"][1]["W"] for p in dps]),
        _cat3([p["layers"][1]["b"][None] for p in dps]),
        _cat3([jnp.broadcast_to(p["prelu"][1], (1, 64)) for p in dps]),
        L2, db2, blk=1024)
    return out[:, :3]


# speedup vs baseline: 1.0225x; 1.0117x over previous
import functools

import jax
import jax.numpy as jnp
from jax import lax
from jax.scipy.linalg import block_diag
from jax.experimental import pallas as pl
from jax.experimental.pallas import tpu as pltpu
from jax.experimental.pallas import tpu_sc as plsc

AX = ("xx", "yy", "zz")
D = 192
F32 = jnp.float32


def _bd3(ws):
    return block_diag(*ws)


def _cat3(bs):
    return jnp.concatenate(bs, axis=-1)


def _dot(a, b):
    return jnp.dot(a, b, preferred_element_type=F32,
                   precision=lax.Precision.DEFAULT)


def _mlp_prelu_jnp(p, x):
    for i in range(2):
        x = x @ p["layers"][i]["W"] + p["layers"][i]["b"]
        a = p["prelu"][i]
        x = jnp.where(x >= 0, x, a * x)
    return x @ p["layers"][2]["W"] + p["layers"][2]["b"]


def _full_spec(shape):
    return pl.BlockSpec(shape, lambda *_: tuple(0 for _ in shape))


def _enc_mlp(x, W0, b0, a0, W1, b1, a1, W2, b2, blk, tile3):
    R, din = x.shape
    dout = W2.shape[1] * (3 if tile3 else 1)

    def body(xr, W0r, b0r, a0r, W1r, b1r, a1r, W2r, b2r, outr):
        t = _dot(xr[...], W0r[...]) + b0r[...]
        t = jnp.where(t >= 0, t, a0r[...] * t)
        t = _dot(t, W1r[...]) + b1r[...]
        t = jnp.where(t >= 0, t, a1r[...] * t)
        h = _dot(t, W2r[...]) + b2r[...]
        if tile3:
            h = jnp.concatenate([h, h, h], axis=1)
        outr[...] = h

    grid = (R // blk,)
    return pl.pallas_call(
        body,
        grid=grid,
        in_specs=[pl.BlockSpec((blk, din), lambda n: (n, 0))]
        + [_full_spec(w.shape) for w in (W0, b0, a0, W1, b1, a1, W2, b2)],
        out_specs=pl.BlockSpec((blk, dout), lambda n: (n, 0)),
        out_shape=jax.ShapeDtypeStruct((R, dout), F32),
    )(x, W0, b0, a0, W1, b1, a1, W2, b2)


def _node_pre(h, vcat, Wsnd, Wrcv, blk):
    N = h.shape[0]

    def body(hr, vr, Wsr, Wrr, hpo, p2o):
        hp = hr[...] + vr[...]
        hpo[...] = hp
        p2o[0] = _dot(hp, Wsr[...])
        p2o[1] = _dot(hp, Wrr[...])

    return pl.pallas_call(
        body,
        grid=(N // blk,),
        in_specs=[
            pl.BlockSpec((blk, D), lambda n: (n, 0)),
            _full_spec((1, D)),
            _full_spec((D, D)),
            _full_spec((D, D)),
        ],
        out_specs=[
            pl.BlockSpec((blk, D), lambda n: (n, 0)),
            pl.BlockSpec((2, blk, D), lambda n: (0, n, 0)),
        ],
        out_shape=[
            jax.ShapeDtypeStruct((N, D), F32),
            jax.ShapeDtypeStruct((2, N, D), F32),
        ],
    )(h, vcat, Wsnd, Wrcv)


def _edge_step(pre2, ea, W1e, b1, W2, b2, W3, b3, first, blk, enc=None):
    E = pre2.shape[1]
    ea_w = ea.shape[1]
    enc = enc or []

    def body(p2r, ear, W1r, b1r, W2r, b2r, W3r, b3r, *rest):
        if first:
            (EW0, eb0, al0, EW1, eb1, al1, EW2, eb2), outs = rest[:8], rest[8:]
            t = _dot(ear[...], EW0[...]) + eb0[...]
            t = jnp.where(t >= 0, t, al0[...] * t)
            t = _dot(t, EW1[...]) + eb1[...]
            t = jnp.where(t >= 0, t, al1[...] * t)
            e64 = _dot(t, EW2[...]) + eb2[...]
            eac = jnp.concatenate([e64, e64, e64], axis=1)
        else:
            outs = rest
            eac = ear[...]
        z = (_dot(eac, W1r[...]) + p2r[0].astype(F32) + p2r[1].astype(F32)
             + b1r[...])
        z = jnp.maximum(z, 0.0)
        z = jnp.maximum(_dot(z, W2r[...]) + b2r[...], 0.0)
        m = _dot(z, W3r[...]) + b3r[...]
        outs[0][...] = m
        if first:
            outs[1][...] = eac + m

    if first:
        out_specs = [pl.BlockSpec((blk, D), lambda n: (n, 0)),
                     pl.BlockSpec((blk, D), lambda n: (n, 0))]
        out_shape = [jax.ShapeDtypeStruct((E, D), F32),
                     jax.ShapeDtypeStruct((E, D), F32)]
    else:
        out_specs = pl.BlockSpec((blk, D), lambda n: (n, 0))
        out_shape = jax.ShapeDtypeStruct((E, D), F32)
    return pl.pallas_call(
        body,
        grid=(E // blk,),
        in_specs=[
            pl.BlockSpec((2, blk, D), lambda n: (0, n, 0)),
            pl.BlockSpec((blk, ea_w), lambda n: (n, 0)),
            _full_spec((D, D)),
            _full_spec((1, D)),
            _full_spec((D, D)),
            _full_spec((1, D)),
            _full_spec((D, D)),
            _full_spec((1, D)),
        ] + [_full_spec(w.shape) for w in enc],
        out_specs=out_specs,
        out_shape=out_shape,
    )(pre2, ea, W1e, b1, W2, b2, W3, b3, *enc)


def _node_update(hp, acc0, G1h, gub, G2, gb2, G3, gb3, N1h, N1r, N1u,
                 nb1, N2, nb2, N3, nb3, blk):
    N = hp.shape[0]

    def body(hpr, acc0r, G1hr, gubr, G2r, gb2r, G3r, gb3r, N1hr, N1rr,
             N1ur, nb1r, N2r, nb2r, N3r, nb3r, hno, poo):
        pid = pl.program_id(0)
        hpv = hpr[...]
        rm = (acc0r[0, 0] - acc0r[0, 1]) + (acc0r[1, 0] - acc0r[1, 1])
        zg = jnp.maximum(_dot(hpv, G1hr[...]) + gubr[...], 0.0)
        zg = jnp.maximum(_dot(zg, G2r[...]) + gb2r[...], 0.0)
        mu = _dot(zg, G3r[...]) + gb3r[...]
        x1 = _dot(hpv, N1hr[...]) + _dot(rm, N1rr[...]) + _dot(mu, N1ur[...])
        x1 = jnp.maximum(x1 + nb1r[...], 0.0)
        x2 = jnp.maximum(_dot(x1, N2r[...]) + nb2r[...], 0.0)
        hn = hpv + _dot(x2, N3r[...]) + nb3r[...]
        hno[...] = hn
        part = jnp.sum(hn, axis=0, keepdims=True)

        @pl.when(pid == 0)
        def _():
            poo[...] = part

        @pl.when(pid != 0)
        def _():
            poo[...] = poo[...] + part

    return pl.pallas_call(
        body,
        grid=(N // blk,),
        in_specs=[
            pl.BlockSpec((blk, D), lambda n: (n, 0)),
            pl.BlockSpec((2, 2, blk, D), lambda n: (0, 0, n, 0)),
            _full_spec((D, D)),
            _full_spec((1, D)),
            _full_spec((D, D)),
            _full_spec((1, D)),
            _full_spec((D, D)),
            _full_spec((1, D)),
            _full_spec((D, D)),
            _full_spec((D, D)),
            _full_spec((D, D)),
            _full_spec((1, D)),
            _full_spec((D, D)),
            _full_spec((1, D)),
            _full_spec((D, D)),
            _full_spec((1, D)),
        ],
        out_specs=[
            pl.BlockSpec((blk, D), lambda n: (n, 0)),
            pl.BlockSpec((1, D), lambda n: (0, 0)),
        ],
        out_shape=[
            jax.ShapeDtypeStruct((N, D), F32),
            jax.ShapeDtypeStruct((1, D), F32),
        ],
    )(hp, acc0, G1h, gub, G2, gb2, G3, gb3, N1h, N1r, N1u, nb1, N2,
      nb2, N3, nb3)


def _decoder(hr, L0, db0, al0, L1, db1, al1, L2, db2, blk):
    M = hr.shape[0]
    dout = L2.shape[1]

    def body(hrr, L0r, db0r, al0r, L1r, db1r, al1r, L2r, db2r, outr):
        t = _dot(hrr[...], L0r[...]) + db0r[...]
        t = jnp.where(t >= 0, t, al0r[...] * t)
        t = _dot(t, L1r[...]) + db1r[...]
        t = jnp.where(t >= 0, t, al1r[...] * t)
        outr[...] = _dot(t, L2r[...]) + db2r[...]

    return pl.pallas_call(
        body,
        grid=(M // blk,),
        in_specs=[pl.BlockSpec((blk, D), lambda n: (n, 0))]
        + [_full_spec(w.shape) for w in (L0, db0, al0, L1, db1, al1, L2, db2)],
        out_specs=pl.BlockSpec((blk, dout), lambda n: (n, 0)),
        out_shape=jax.ShapeDtypeStruct((M, dout), F32),
    )(hr, L0, db0, al0, L1, db1, al1, L2, db2)


def _sc_gather(table, idx2d, chunk):
    n_ch_tot, chunk_ = idx2d.shape
    assert chunk_ == chunk
    M = n_ch_tot * chunk
    width = table.shape[1]
    nw = 32
    per_w = M // nw
    n_chunks = per_w // chunk
    assert per_w * nw == M and n_chunks * chunk == per_w and n_chunks >= 4
    NB = 4
    nquads = (n_chunks // NB) * NB
    tail = n_chunks - nquads
    mesh = plsc.VectorSubcoreMesh(core_axis_name="c", subcore_axis_name="s")

    @functools.partial(
        pl.kernel,
        mesh=mesh,
        out_type=jax.ShapeDtypeStruct((M, width), table.dtype),
        compiler_params=pltpu.CompilerParams(use_tc_tiling_on_sc=False),
        scratch_types=[
            pltpu.VMEM((n_chunks, chunk), jnp.int32),
        ]
        + [pltpu.VMEM((chunk, width), table.dtype) for _ in range(NB)]
        + [pltpu.SemaphoreType.DMA for _ in range(2 * NB)],
    )
    def k(t_hbm, i_hbm, o_hbm, idx_v, *bufs_sems):
        rbuf = bufs_sems[:NB]
        gs = bufs_sems[NB:2 * NB]
        ws = bufs_sems[2 * NB:]
        wid = lax.axis_index("s") * 2 + lax.axis_index("c")
        base = wid * per_w
        c0 = wid * n_chunks
        pltpu.sync_copy(i_hbm.at[pl.ds(c0, n_chunks)], idx_v)
        for j in range(NB):
            pltpu.async_copy(t_hbm.at[idx_v.at[j]], rbuf[j], gs[j])

        @pl.loop(0, nquads, step=NB)
        def _(ci):
            for j in range(NB):
                off = base + (ci + j) * chunk
                pltpu.make_async_copy(
                    t_hbm.at[idx_v.at[ci + j]], rbuf[j], gs[j]).wait()
                pltpu.async_copy(rbuf[j], o_hbm.at[pl.ds(off, chunk)], ws[j])
            for j in range(NB):
                off = base + (ci + j) * chunk
                pltpu.make_async_copy(
                    rbuf[j], o_hbm.at[pl.ds(off, chunk)], ws[j]).wait()

                @pl.when(ci + NB + j < n_chunks)
                def _():
                    pltpu.async_copy(
                        t_hbm.at[idx_v.at[ci + NB + j]], rbuf[j], gs[j])

        for j in range(tail):
            cj = nquads + j
            off = base + cj * chunk
            pltpu.make_async_copy(
                t_hbm.at[idx_v.at[cj]], rbuf[j], gs[j]).wait()
            pltpu.sync_copy(rbuf[j], o_hbm.at[pl.ds(off, chunk)])

    return k(table, idx2d)


def _sc_scatter(m0, m1, idx3, zeros, chunk):
    E = 2 * m0.shape[0]
    N = zeros.shape[0]
    DH = D // 3
    n_sub = 16
    per_t = E // (2 * n_sub)
    n_chunks = per_t // chunk
    rpt = N // n_sub
    assert per_t * 2 * n_sub == E and n_chunks * chunk == per_t
    assert rpt * n_sub == N and zeros.shape[1] == DH
    NB = 4
    nquads = (n_chunks // NB) * NB
    tail = n_chunks - nquads
    mesh = plsc.VectorSubcoreMesh(core_axis_name="c", subcore_axis_name="s")

    @functools.partial(
        pl.kernel,
        mesh=mesh,
        out_type=jax.ShapeDtypeStruct((2, 2, N, D), F32),
        compiler_params=pltpu.CompilerParams(use_tc_tiling_on_sc=False),
        scratch_types=[
            pltpu.VMEM((n_chunks, chunk), jnp.int32),
            pltpu.VMEM((n_chunks, chunk), jnp.int32),
            pltpu.VMEM_SHARED((N, DH), F32),
            pltpu.VMEM_SHARED((N, DH), F32),
        ]
        + [pltpu.VMEM((chunk, DH), F32) for _ in range(NB)]
        + [pltpu.SemaphoreType.DMA for _ in range(NB)],
    )
    def k(m0_hbm, m1_hbm, i_hbm, z_hbm, o_hbm, idxR, idxS, accP, accM,
          *bufs_sems):
        mb = bufs_sems[:NB]
        ls = bufs_sems[NB:]
        c = lax.axis_index("c")
        s = lax.axis_index("s")
        tile_e0 = s * per_t
        tile_c0 = c * (E // (2 * chunk)) + s * n_chunks
        pltpu.sync_copy(i_hbm.at[0, pl.ds(tile_c0, n_chunks)], idxR)
        pltpu.sync_copy(i_hbm.at[1, pl.ds(tile_c0, n_chunks)], idxS)

        def pipeline(m_hbm):
            def load(ci, buf, sem):
                pltpu.async_copy(
                    m_hbm.at[pl.ds(tile_e0 + ci * chunk, chunk),
                             pl.ds(col, DH)], buf, sem)

            for cp in range(3):
                col = cp * DH
                pltpu.sync_copy(z_hbm.at[pl.ds(s * rpt, rpt)],
                                accP.at[pl.ds(s * rpt, rpt)])
                pltpu.sync_copy(z_hbm.at[pl.ds(s * rpt, rpt)],
                                accM.at[pl.ds(s * rpt, rpt)])
                plsc.subcore_barrier()
                for j in range(NB):
                    load(j, mb[j], ls[j])

                @pl.loop(0, nquads, step=NB)
                def _(ci):
                    for j in range(NB):
                        pltpu.make_async_copy(
                            m_hbm.at[pl.ds(tile_e0 + (ci + j) * chunk, chunk),
                                     pl.ds(col, DH)], mb[j], ls[j]).wait()
                        pltpu.sync_copy(mb[j], accP.at[idxR.at[ci + j]],
                                        add=True)
                        pltpu.sync_copy(mb[j], accM.at[idxS.at[ci + j]],
                                        add=True)

                        @pl.when(ci + NB + j < n_chunks)
                        def _():
                            load(ci + NB + j, mb[j], ls[j])

                for j in range(tail):
                    cj = nquads + j
                    pltpu.make_async_copy(
                        m_hbm.at[pl.ds(tile_e0 + cj * chunk, chunk),
                                 pl.ds(col, DH)], mb[j], ls[j]).wait()
                    pltpu.sync_copy(mb[j], accP.at[idxR.at[cj]], add=True)
                    pltpu.sync_copy(mb[j], accM.at[idxS.at[cj]], add=True)

                plsc.subcore_barrier()
                pltpu.sync_copy(accP.at[pl.ds(s * rpt, rpt)],
                                o_hbm.at[c, 0, pl.ds(s * rpt, rpt),
                                         pl.ds(col, DH)])
                pltpu.sync_copy(accM.at[pl.ds(s * rpt, rpt)],
                                o_hbm.at[c, 1, pl.ds(s * rpt, rpt),
                                         pl.ds(col, DH)])

        @pl.when(c == 0)
        def _():
            pipeline(m0_hbm)

        @pl.when(c == 1)
        def _():
            pipeline(m1_hbm)

    return k(m0, m1, idx3, zeros)


def _step_weights(params, uenc, i):
    W1 = [params[a]["mp"][i]["edge"][0]["W"] for a in AX]
    G1 = [params[a]["mp"][i]["glob"][0]["W"] for a in AX]
    Nw1 = [params[a]["mp"][i]["node"][0]["W"] for a in AX]
    gub = _cat3([
        uenc @ G1[k][64:128] + params[AX[k]]["mp"][i]["glob"][0]["b"][None]
        for k in range(3)
    ])
    return dict(
        W1e=_bd3([w[0:64] for w in W1]),
        Wsnd=_bd3([w[64:128] for w in W1]),
        Wrcv=_bd3([w[128:192] for w in W1]),
        b1=_cat3([params[a]["mp"][i]["edge"][0]["b"][None] for a in AX]),
        W2=_bd3([params[a]["mp"][i]["edge"][1]["W"] for a in AX]),
        b2=_cat3([params[a]["mp"][i]["edge"][1]["b"][None] for a in AX]),
        W3=_bd3([params[a]["mp"][i]["edge"][2]["W"] for a in AX]),
        b3=_cat3([params[a]["mp"][i]["edge"][2]["b"][None] for a in AX]),
        G1h=_bd3([w[0:64] for w in G1]),
        gub=gub,
        G2=_bd3([params[a]["mp"][i]["glob"][1]["W"] for a in AX]),
        gb2=_cat3([params[a]["mp"][i]["glob"][1]["b"][None] for a in AX]),
        G3=_bd3([params[a]["mp"][i]["glob"][2]["W"] for a in AX]),
        gb3=_cat3([params[a]["mp"][i]["glob"][2]["b"][None] for a in AX]),
        N1h=_bd3([w[0:64] for w in Nw1]),
        N1r=_bd3([w[64:128] for w in Nw1]),
        N1u=_bd3([w[128:192] for w in Nw1]),
        nb1=_cat3([params[a]["mp"][i]["node"][0]["b"][None] for a in AX]),
        N2=_bd3([params[a]["mp"][i]["node"][1]["W"] for a in AX]),
        nb2=_cat3([params[a]["mp"][i]["node"][1]["b"][None] for a in AX]),
        N3=_bd3([params[a]["mp"][i]["node"][2]["W"] for a in AX]),
        nb3=_cat3([params[a]["mp"][i]["node"][2]["b"][None] for a in AX]),
    )


def kernel(x, edge_attr, u, params, edge_index, batch, real_nodes):
    N = x.shape[0]
    E = edge_index.shape[1]
    recv = edge_index[0]
    send = edge_index[1]

    pe = params["node_enc"]
    h = _enc_mlp(
        x,
        pe["layers"][0]["W"], pe["layers"][0]["b"][None],
        jnp.broadcast_to(pe["prelu"][0], (1, 64)),
        pe["layers"][1]["W"], pe["layers"][1]["b"][None],
        jnp.broadcast_to(pe["prelu"][1], (1, 64)),
        pe["layers"][2]["W"], pe["layers"][2]["b"][None],
        blk=2000, tile3=True)
    pe = params["edge_enc"]
    enc_w = [
        pe["layers"][0]["W"], pe["layers"][0]["b"][None],
        jnp.broadcast_to(pe["prelu"][0], (1, 64)),
        pe["layers"][1]["W"], pe["layers"][1]["b"][None],
        jnp.broadcast_to(pe["prelu"][1], (1, 64)),
        pe["layers"][2]["W"], pe["layers"][2]["b"][None],
    ]
    uenc = _mlp_prelu_jnp(params["u_enc"], u)

    v = _cat3([params[a]["vn"][0]["emb"][None, :] for a in AX])
    EH = E // 2
    idx_g = [
        jnp.concatenate([send[h * EH:(h + 1) * EH],
                         recv[h * EH:(h + 1) * EH] + N]).reshape(-1, 80)
        for h in range(2)
    ]
    idx_sc = jnp.stack([recv, send]).reshape(2, -1, 80)
    zeros = jnp.zeros((N, D // 3), F32)

    ea = [None, None]
    for i in range(2):
        w = _step_weights(params, uenc, i)
        hp, P2 = _node_pre(h, v, w["Wsnd"], w["Wrcv"], blk=2000)
        table = P2.reshape(2 * N, D)
        msgs2 = []
        for hh in range(2):
            pre2 = _sc_gather(table, idx_g[hh], chunk=80).reshape(2, EH, D)
            if i == 0:
                eah = lax.dynamic_slice_in_dim(edge_attr, hh * EH, EH, axis=0)
                msgs, ea[hh] = _edge_step(pre2, eah, w["W1e"], w["b1"],
                                          w["W2"], w["b2"], w["W3"], w["b3"],
                                          first=True, blk=4000, enc=enc_w)
            else:
                msgs = _edge_step(pre2, ea[hh], w["W1e"], w["b1"], w["W2"],
                                  w["b2"], w["W3"], w["b3"], first=False,
                                  blk=4000)
            msgs2.append(msgs)
        acc = _sc_scatter(msgs2[0], msgs2[1], idx_sc, zeros, chunk=80)
        h, pooled = _node_update(hp, acc, w["G1h"], w["gub"],
                                 w["G2"], w["gb2"], w["G3"], w["gb3"],
                                 w["N1h"], w["N1r"], w["N1u"], w["nb1"],
                                 w["N2"], w["nb2"], w["N3"], w["nb3"],
                                 blk=2000)
        vnext = []
        for k, a in enumerate(AX):
            vp = params[a]["vn"][i]
            t = pooled[:, 64 * k:64 * (k + 1)] + v[:, 64 * k:64 * (k + 1)]
            t = jnp.maximum(t @ vp["mlp"][0]["W"] + vp["mlp"][0]["b"], 0.0)
            t = jnp.maximum(t @ vp["mlp"][1]["W"] + vp["mlp"][1]["b"], 0.0)
            vnext.append(t)
        v = _cat3(vnext)

    hr = _sc_gather(h, real_nodes.reshape(-1, 64), chunk=64)
    dps = [params[n] for n in ("dx", "dy", "dz")]
    L2 = jnp.zeros((D, 128), F32)
    db2 = jnp.zeros((1, 128), F32)
    for k, p in enumerate(dps):
        L2 = L2.at[64 * k:64 * (k + 1), k].set(p["layers"][2]["W"][:, 0])
        db2 = db2.at[0, k].set(p["layers"][2]["b"][0])
    out = _decoder(
        hr,
        _bd3([p["layers"][0]["W"] for p in dps]),
        _cat3([p["layers"][0]["b"][None] for p in dps]),
        _cat3([jnp.broadcast_to(p["prelu"][0], (1, 64)) for p in dps]),
        _bd3([p["layers"][1]["W"] for p in dps]),
        _cat3([p["layers"][1]["b"][None] for p in dps]),
        _cat3([jnp.broadcast_to(p["prelu"][1], (1, 64)) for p in dps]),
        L2, db2, blk=1024)
    return out[:, :3]
